# Initial kernel scaffold; baseline (speedup 1.0000x reference)
import jax
import jax.numpy as jnp
from jax.experimental import pallas as pl


def kernel(x, params, edge_index, broadcast, edge_index_hr):
    raise NotImplementedError("write your pallas kernel here")


# speedup vs baseline: 34.8671x; 34.8671x over previous
import functools

import jax
import jax.numpy as jnp
from jax import lax
from jax.experimental import pallas as pl
from jax.experimental.pallas import tpu as pltpu
from jax.experimental.pallas import tpu_sc as plsc

N = 100000
NP = 100352
NHR = 200000
NHRP = 200704
E = 1600000
EHR = 3200000
EP = 1638400
EPH = 3211264
NC, NS = 2, 16
ZR = 784
BR = 1024

_I32_MIN = -2147483648


@functools.lru_cache(None)
def _edge_agg(np_rows, f, nblocks):
    rows_per_tile = np_rows // NS
    nz = rows_per_tile // ZR
    mesh = plsc.VectorSubcoreMesh(
        core_axis_name="c", subcore_axis_name="s", num_cores=NC, num_subcores=NS
    )

    one_d = f == 1

    def body(src_hbm, dst_hbm, table_hbm, zrow_hbm, out_hbm,
             acc, idx_v, rows_v, gsem, ssem, isem):
        c = lax.axis_index("c")
        s = lax.axis_index("s")
        wid = c * NS + s
        for i in range(nz):
            if one_d:
                pltpu.sync_copy(zrow_hbm,
                                acc.at[pl.ds(s * rows_per_tile + i * ZR, ZR)])
            else:
                pltpu.sync_copy(zrow_hbm,
                                acc.at[pl.ds(s * rows_per_tile + i * ZR, ZR), :])
        plsc.subcore_barrier()

        base = wid * nblocks * 8

        def start_idx(b, p):
            pltpu.async_copy(src_hbm.at[pl.ds(base + b * 8, 8)], idx_v.at[p, 0],
                             isem.at[p, 0])
            pltpu.async_copy(dst_hbm.at[pl.ds(base + b * 8, 8)], idx_v.at[p, 1],
                             isem.at[p, 1])

        def wait_idx(p):
            pltpu.make_async_copy(src_hbm.at[pl.ds(0, 8)], idx_v.at[p, 0],
                                  isem.at[p, 0]).wait()
            pltpu.make_async_copy(dst_hbm.at[pl.ds(0, 8)], idx_v.at[p, 1],
                                  isem.at[p, 1]).wait()

        def do_block(b, p):
            wait_idx(p)

            @pl.when(b + 1 < nblocks)
            def _():
                start_idx(b + 1, 1 - p)

            gd = [None] * 8
            sd = [None] * 8
            for j in range(4):
                gd[j] = pltpu.async_copy(table_hbm.at[idx_v.at[p, 0, j]],
                                         rows_v.at[j], gsem.at[j])
            for j in range(4):
                gd[j].wait()
                sd[j] = pltpu.async_copy(rows_v.at[j],
                                         acc.at[idx_v.at[p, 1, j]],
                                         ssem.at[j], add=True)
            for j in range(4, 8):
                sd[j - 4].wait()
                gd[j] = pltpu.async_copy(table_hbm.at[idx_v.at[p, 0, j]],
                                         rows_v.at[j - 4], gsem.at[j - 4])
            for j in range(4, 8):
                gd[j].wait()
                sd[j] = pltpu.async_copy(rows_v.at[j - 4],
                                         acc.at[idx_v.at[p, 1, j]],
                                         ssem.at[j - 4], add=True)
            for j in range(4, 8):
                sd[j].wait()

        start_idx(0, 0)

        def loop_body(i, carry):
            b = i * 2
            do_block(b, 0)

            @pl.when(b + 1 < nblocks)
            def _():
                do_block(b + 1, 1)

            return carry

        lax.fori_loop(0, (nblocks + 1) // 2, loop_body, jnp.int32(0))

        plsc.subcore_barrier()
        if one_d:
            pltpu.sync_copy(acc.at[pl.ds(s * rows_per_tile, rows_per_tile)],
                            out_hbm.at[c, pl.ds(s * rows_per_tile, rows_per_tile)])
        else:
            pltpu.sync_copy(acc.at[pl.ds(s * rows_per_tile, rows_per_tile), :],
                            out_hbm.at[c, pl.ds(s * rows_per_tile, rows_per_tile), :])

    out_sh = (NC, np_rows) if one_d else (NC, np_rows, f)
    acc_sh = (np_rows,) if one_d else (np_rows, f)
    rows_sh = (4, 128) if one_d else (4, 128, f)
    return pl.kernel(
        body,
        out_type=jax.ShapeDtypeStruct(out_sh, jnp.float32),
        mesh=mesh,
        compiler_params=pltpu.CompilerParams(use_tc_tiling_on_sc=False),
        scratch_types=[
            pltpu.VMEM_SHARED(acc_sh, jnp.float32),
            pltpu.VMEM((2, 2, 8, 128), jnp.int32),
            pltpu.VMEM(rows_sh, jnp.float32),
            pltpu.SemaphoreType.DMA((4,)),
            pltpu.SemaphoreType.DMA((4,)),
            pltpu.SemaphoreType.DMA((2, 2)),
        ],
    )


def _agg(src2d, dst2d, table, np_rows, f, nblocks):
    if f > 16:
        parts = [_agg(src2d, dst2d, table[:, :16], np_rows, 16, nblocks)]
        for j in range(16, f):
            parts.append(_agg(src2d, dst2d, table[:, j:j + 1], np_rows, 1,
                              nblocks))
        return jnp.concatenate(parts, axis=2)
    if f == 1:
        z = jnp.zeros((ZR,), jnp.float32)
        out = _edge_agg(np_rows, 1, nblocks)(src2d, dst2d,
                                             table.reshape(np_rows), z)
        return out.reshape(NC, np_rows, 1)
    z = jnp.zeros((ZR, f), jnp.float32)
    return _edge_agg(np_rows, f, nblocks)(src2d, dst2d, table, z)


def _mm(x, w):
    if w.shape[0] == 1:
        return x * w
    return jnp.dot(x, w, preferred_element_type=jnp.float32)


def _rowcall(fn, np_rows, n_row_args, out_widths, *args):
    row_args = args[:n_row_args]
    const_args = args[n_row_args:]
    grid = np_rows // BR
    in_specs = (
        [pl.BlockSpec((BR, a.shape[1]), lambda i: (i, 0)) for a in row_args]
        + [pl.BlockSpec(c.shape, lambda i: (0, 0)) for c in const_args]
    )
    out_specs = [pl.BlockSpec((BR, w), lambda i: (i, 0)) for w in out_widths]
    out_shape = [jax.ShapeDtypeStruct((np_rows, w), jnp.float32) for w in out_widths]

    def kfn(*refs):
        nin = len(args)
        vals = [r[...] for r in refs[:nin]]
        outs = fn(*vals)
        if len(out_widths) == 1:
            outs = (outs,)
        for r, o in zip(refs[nin:], outs):
            r[...] = o

    res = pl.pallas_call(kfn, grid=(grid,), in_specs=in_specs,
                         out_specs=out_specs, out_shape=out_shape)(*args)
    return res[0] if len(out_widths) == 1 else res


def _topk_mask(score2d, act2d, kk):
    rows = score2d.shape[0]

    def kfn(s_ref, a_ref, o_ref):
        sc = s_ref[...]
        a = a_ref[...]
        b = lax.bitcast_convert_type(sc, jnp.int32)
        key = jnp.where(b < 0, jnp.bitwise_xor(b, jnp.int32(0x7FFFFFFF)), b)
        key = jnp.where(a > 0, key, _I32_MIN)

        def step(i, cu):
            bit = jnp.left_shift(jnp.int32(1), 31 - i)
            c2 = jnp.bitwise_or(cu, bit)
            thr = jnp.bitwise_xor(c2, _I32_MIN)
            cnt = jnp.sum((key >= thr).astype(jnp.int32))
            return jnp.where(cnt >= kk, c2, cu)

        cu = lax.fori_loop(0, 32, step, jnp.int32(0))
        thr = jnp.bitwise_xor(cu, _I32_MIN)
        o_ref[...] = ((key >= thr) & (a > 0)).astype(jnp.float32)

    return pl.pallas_call(
        kfn,
        out_shape=jax.ShapeDtypeStruct((rows, 128), jnp.float32),
    )(score2d, act2d)


def _conv_finish(p0, p1, si, act, w, bvec, pn, fi):
    def fn(a0, a1, sv, av, wv, bv, pv):
        agg = (a0 + a1)[:, :fi] * sv
        h = (_mm(agg, wv) + bv) * av
        y = _mm(h, pv)
        return h, y

    fo = w.shape[1]
    return _rowcall(fn, NP, 4, [fo, fo], p0, p1, si, act, w, bvec, pn)


def _gate_table(h, y, actn, dgo0, dgo1, dgi0, dgi1, fpad):
    fo = h.shape[1]

    def fn(hv, yv, av, o0, o1, i0, i1):
        so = lax.rsqrt(jnp.maximum(o0 + o1, 1.0))
        si = lax.rsqrt(jnp.maximum(i0 + i1, 1.0))
        t = hv * jax.nn.sigmoid(yv) * (av * so)
        if fpad > fo:
            t = jnp.concatenate(
                [t, jnp.zeros((t.shape[0], fpad - fo), jnp.float32)], axis=1)
        return t, so, si

    return _rowcall(fn, NP, 7, [fpad, 1, 1], h, y, actn, dgo0, dgo1, dgi0, dgi1)


def _up_step(p0, p1, si, act, bvec, skip, so_next, w_next, fi, fpad):
    def fn(a0, a1, sv, av, hsv, sn, bv, wn):
        agg = (a0 + a1)[:, :fi] * sv
        h = (agg + bv) * av
        u = (h + hsv) * sn
        t = _mm(u, wn)
        if fpad > t.shape[1]:
            t = jnp.concatenate(
                [t, jnp.zeros((t.shape[0], fpad - t.shape[1]), jnp.float32)],
                axis=1)
        return t

    return _rowcall(fn, NP, 6, [fpad], p0, p1, si, act, skip, so_next,
                    bvec, w_next)


def _prep_edges(edge_index, e, n, np_rows, ep):
    pad = ep - e
    pidx = (n + (jnp.arange(pad, dtype=jnp.int32) % (np_rows - n))).astype(jnp.int32)
    s2 = jnp.concatenate([edge_index[0], pidx]).reshape(ep // 128, 128)
    d2 = jnp.concatenate([edge_index[1], pidx]).reshape(ep // 128, 128)
    return s2, d2


def _to2d(col):
    return col.reshape(col.shape[0] // 128, 128)


def _tocol(arr2d):
    return arr2d.reshape(arr2d.shape[0] * 128, 1)


def kernel(x, params, edge_index, broadcast, edge_index_hr):
    del broadcast
    p = params
    src2, dst2 = _prep_edges(edge_index, E, N, NP, EP)
    hsrc2, hdst2 = _prep_edges(edge_index_hr, EHR, NHR, NHRP, EPH)

    act0 = (jnp.arange(NP) < N).astype(jnp.float32).reshape(NP, 1)
    acthr = (jnp.arange(NHRP) < NHR).astype(jnp.float32).reshape(NHRP, 1)
    xp = jnp.pad(x, ((0, NP - N), (0, 0)))

    def pnorm(mat):
        return mat * lax.rsqrt(jnp.sum(mat * mat))

    ks = [75000, 56250, 42187, 31640]

    d0i = _agg(src2, dst2, act0, NP, 1, 50)
    d0o = _agg(dst2, src2, act0, NP, 1, 50)

    def t1fn(xv, o0, o1, i0, i1):
        so = lax.rsqrt(jnp.maximum(o0 + o1, 1.0))
        si = lax.rsqrt(jnp.maximum(i0 + i1, 1.0))
        return xv * so, so, si

    table1, so0, si0 = _rowcall(t1fn, NP, 5, [1, 1, 1],
                                xp, d0o[0], d0o[1], d0i[0], d0i[1])
    a1 = _agg(src2, dst2, table1, NP, 1, 50)
    h0, y1 = _conv_finish(a1[0], a1[1], si0, act0, p["W_c1"],
                          p["b_c1"].reshape(1, -1), pnorm(p["p1"]), 1)

    hs, ys, acts, sos, sis = [h0], [y1], [act0], [so0], [si0]
    convs = [("c2", 1, 10, 16), ("c3", 10, 14, 16), ("c4", 14, 18, 16),
             ("bn", 18, 18, 18)]
    pools = ["p2", "p3", "p4", None]
    for lvl in range(4):
        actn2d = _topk_mask(_to2d(ys[-1][:, 0:1].reshape(NP)),
                            _to2d(acts[-1].reshape(NP)), ks[lvl])
        actn = _tocol(actn2d)
        dni = _agg(src2, dst2, actn, NP, 1, 50)
        dno = _agg(dst2, src2, actn, NP, 1, 50)
        fo_prev = hs[-1].shape[1]
        fpad = 1 if fo_prev == 1 else (16 if fo_prev <= 16 else 18)
        table, so, si = _gate_table(hs[-1], ys[-1], actn,
                                    dno[0], dno[1], dni[0], dni[1], fpad)
        ag = _agg(src2, dst2, table, NP, fpad, 50)
        name, fi, fo, _ = convs[lvl]
        pool = pools[lvl]
        pmat = pnorm(p[pool]) if pool else jnp.eye(fo, dtype=jnp.float32)
        h, y = _conv_finish(ag[0], ag[1], si, actn, p["W_" + name],
                            p["b_" + name].reshape(1, -1), pmat, fi)
        acts.append(actn)
        sos.append(so)
        sis.append(si)
        hs.append(h)
        ys.append(y)

    def mkfn(fo_t, fpad_t):
        def fn(hbnv, hsv, snv, wnv):
            u = (hbnv + hsv) * snv
            t = _mm(u, wnv)
            if fpad_t > t.shape[1]:
                t = jnp.concatenate(
                    [t, jnp.zeros((t.shape[0], fpad_t - t.shape[1]),
                                  jnp.float32)], axis=1)
            return t
        return fn

    tab = _rowcall(mkfn(14, 16), NP, 3, [16], hs[4], hs[3], sos[3], p["W_u1"])
    ag = _agg(src2, dst2, tab, NP, 16, 50)
    tab = _up_step(ag[0], ag[1], sis[3], acts[3], p["b_u1"].reshape(1, -1),
                   hs[2], sos[2], p["W_u2"], 14, 16)
    ag = _agg(src2, dst2, tab, NP, 16, 50)
    tab = _up_step(ag[0], ag[1], sis[2], acts[2], p["b_u2"].reshape(1, -1),
                   hs[1], sos[1], p["W_u3"], 10, 1)
    ag = _agg(src2, dst2, tab, NP, 1, 50)
    tab = _up_step(ag[0], ag[1], sis[1], acts[1], p["b_u3"].reshape(1, -1),
                   hs[0], sos[0], p["W_u4"], 1, 1)
    ag = _agg(src2, dst2, tab, NP, 1, 50)

    def u4fn(a0, a1, sv, av, bv):
        return (a0 + a1) * sv * av + bv * av

    u4 = _rowcall(u4fn, NP, 4, [1], ag[0], ag[1], si0, act0,
                  p["b_u4"].reshape(1, -1))

    neu = jnp.concatenate([u4[:N], u4[:N]], axis=1).reshape(2 * N, 1)
    neu = jnp.pad(neu, ((0, NHRP - NHR), (0, 0)))

    dhi = _agg(hsrc2, hdst2, acthr, NHRP, 1, 98)
    dho = _agg(hdst2, hsrc2, acthr, NHRP, 1, 98)

    def thrfn(nv, o0, o1, i0, i1):
        so = lax.rsqrt(jnp.maximum(o0 + o1, 1.0))
        si = lax.rsqrt(jnp.maximum(i0 + i1, 1.0))
        return nv * so, si

    tabbc, sihr = _rowcall(thrfn, NHRP, 5, [1, 1],
                           neu, dho[0], dho[1], dhi[0], dhi[1])
    abc = _agg(hsrc2, hdst2, tabbc, NHRP, 1, 98)

    projs = [(p["projW%d" % j], p["projb%d" % j].reshape(1, -1))
             for j in range(6)]

    def headfn(a0, a1, sv, av, wbc, bbc, *wb):
        h = (a0 + a1) * sv * wbc + bbc
        h = h * av
        for j in range(6):
            h = _mm(h, wb[2 * j]) + wb[2 * j + 1]
            if j < 5:
                h = jnp.where(h >= 0, h, -0.8 * h)
                h = jnp.tanh(h)
        return h

    consts = [p["W_bc"], p["b_bc"].reshape(1, -1)]
    for w, b in projs:
        consts.extend([w, b])
    out = _rowcall(headfn, NHRP, 4, [3], abc[0], abc[1], sihr, acthr, *consts)
    return out[:NHR]


# speedup vs baseline: 35.4957x; 1.0180x over previous
import functools

import jax
import jax.numpy as jnp
from jax import lax
from jax.experimental import pallas as pl
from jax.experimental.pallas import tpu as pltpu
from jax.experimental.pallas import tpu_sc as plsc

N = 100000
NP = 100352
NHR = 200000
NHRP = 200704
E = 1600000
EHR = 3200000
EP = 1638400
EPH = 3211264
NC, NS = 2, 16
CH = 1024
ZR = 784
BR = 1024

_I32_MIN = -2147483648


@functools.lru_cache(None)
def _edge_agg(np_rows, f, nblocks):
    rows_per_tile = np_rows // NS
    nz = rows_per_tile // ZR
    mesh = plsc.VectorSubcoreMesh(
        core_axis_name="c", subcore_axis_name="s", num_cores=NC, num_subcores=NS
    )

    one_d = f == 1

    def body(src_hbm, dst_hbm, table_hbm, zrow_hbm, out_hbm,
             acc, idx_v, rows_v, gsem, ssem, isem):
        c = lax.axis_index("c")
        s = lax.axis_index("s")
        wid = c * NS + s
        for i in range(nz):
            if one_d:
                pltpu.sync_copy(zrow_hbm,
                                acc.at[pl.ds(s * rows_per_tile + i * ZR, ZR)])
            else:
                pltpu.sync_copy(zrow_hbm,
                                acc.at[pl.ds(s * rows_per_tile + i * ZR, ZR), :])
        plsc.subcore_barrier()

        base = wid * nblocks * CH

        def start_idx(b, p):
            pltpu.async_copy(src_hbm.at[pl.ds(base + b * CH, CH)],
                             idx_v.at[p, 0], isem.at[p, 0])
            pltpu.async_copy(dst_hbm.at[pl.ds(base + b * CH, CH)],
                             idx_v.at[p, 1], isem.at[p, 1])

        def wait_idx(p):
            pltpu.make_async_copy(src_hbm.at[pl.ds(0, CH)], idx_v.at[p, 0],
                                  isem.at[p, 0]).wait()
            pltpu.make_async_copy(dst_hbm.at[pl.ds(0, CH)], idx_v.at[p, 1],
                                  isem.at[p, 1]).wait()

        def do_block(b, p):
            wait_idx(p)

            @pl.when(b + 1 < nblocks)
            def _():
                start_idx(b + 1, 1 - p)

            pltpu.async_copy(table_hbm.at[idx_v.at[p, 0]], rows_v, gsem).wait()
            pltpu.async_copy(rows_v, acc.at[idx_v.at[p, 1]], ssem,
                             add=True).wait()

        start_idx(0, 0)

        def loop_body(i, carry):
            b = i * 2
            do_block(b, 0)

            @pl.when(b + 1 < nblocks)
            def _():
                do_block(b + 1, 1)

            return carry

        lax.fori_loop(0, (nblocks + 1) // 2, loop_body, jnp.int32(0))

        plsc.subcore_barrier()
        if one_d:
            pltpu.sync_copy(acc.at[pl.ds(s * rows_per_tile, rows_per_tile)],
                            out_hbm.at[c, pl.ds(s * rows_per_tile, rows_per_tile)])
        else:
            pltpu.sync_copy(acc.at[pl.ds(s * rows_per_tile, rows_per_tile), :],
                            out_hbm.at[c, pl.ds(s * rows_per_tile, rows_per_tile), :])

    out_sh = (NC, np_rows) if one_d else (NC, np_rows, f)
    acc_sh = (np_rows,) if one_d else (np_rows, f)
    rows_sh = (CH,) if one_d else (CH, f)
    return pl.kernel(
        body,
        out_type=jax.ShapeDtypeStruct(out_sh, jnp.float32),
        mesh=mesh,
        compiler_params=pltpu.CompilerParams(use_tc_tiling_on_sc=False),
        scratch_types=[
            pltpu.VMEM_SHARED(acc_sh, jnp.float32),
            pltpu.VMEM((2, 2, CH), jnp.int32),
            pltpu.VMEM(rows_sh, jnp.float32),
            pltpu.SemaphoreType.DMA,
            pltpu.SemaphoreType.DMA,
            pltpu.SemaphoreType.DMA((2, 2)),
        ],
    )


def _agg(src2d, dst2d, table, np_rows, f, nblocks):
    if f > 16:
        parts = [_agg(src2d, dst2d, table[:, :16], np_rows, 16, nblocks)]
        for j in range(16, f):
            parts.append(_agg(src2d, dst2d, table[:, j:j + 1], np_rows, 1,
                              nblocks))
        return jnp.concatenate(parts, axis=2)
    if f == 1:
        z = jnp.zeros((ZR,), jnp.float32)
        out = _edge_agg(np_rows, 1, nblocks)(src2d, dst2d,
                                             table.reshape(np_rows), z)
        return out.reshape(NC, np_rows, 1)
    z = jnp.zeros((ZR, f), jnp.float32)
    return _edge_agg(np_rows, f, nblocks)(src2d, dst2d, table, z)


def _mm(x, w):
    if w.shape[0] == 1:
        return x * w
    return jnp.dot(x, w, preferred_element_type=jnp.float32)


def _rowcall(fn, np_rows, n_row_args, out_widths, *args):
    row_args = args[:n_row_args]
    const_args = args[n_row_args:]
    grid = np_rows // BR
    in_specs = (
        [pl.BlockSpec((BR, a.shape[1]), lambda i: (i, 0)) for a in row_args]
        + [pl.BlockSpec(c.shape, lambda i: (0, 0)) for c in const_args]
    )
    out_specs = [pl.BlockSpec((BR, w), lambda i: (i, 0)) for w in out_widths]
    out_shape = [jax.ShapeDtypeStruct((np_rows, w), jnp.float32) for w in out_widths]

    def kfn(*refs):
        nin = len(args)
        vals = [r[...] for r in refs[:nin]]
        outs = fn(*vals)
        if len(out_widths) == 1:
            outs = (outs,)
        for r, o in zip(refs[nin:], outs):
            r[...] = o

    res = pl.pallas_call(kfn, grid=(grid,), in_specs=in_specs,
                         out_specs=out_specs, out_shape=out_shape)(*args)
    return res[0] if len(out_widths) == 1 else res


def _topk_mask(score2d, act2d, kk):
    rows = score2d.shape[0]

    def kfn(s_ref, a_ref, o_ref):
        sc = s_ref[...]
        a = a_ref[...]
        b = lax.bitcast_convert_type(sc, jnp.int32)
        key = jnp.where(b < 0, jnp.bitwise_xor(b, jnp.int32(0x7FFFFFFF)), b)
        key = jnp.where(a > 0, key, _I32_MIN)

        def step(i, cu):
            bit = jnp.left_shift(jnp.int32(1), 31 - i)
            c2 = jnp.bitwise_or(cu, bit)
            thr = jnp.bitwise_xor(c2, _I32_MIN)
            cnt = jnp.sum((key >= thr).astype(jnp.int32))
            return jnp.where(cnt >= kk, c2, cu)

        cu = lax.fori_loop(0, 32, step, jnp.int32(0))
        thr = jnp.bitwise_xor(cu, _I32_MIN)
        o_ref[...] = ((key >= thr) & (a > 0)).astype(jnp.float32)

    return pl.pallas_call(
        kfn,
        out_shape=jax.ShapeDtypeStruct((rows, 128), jnp.float32),
    )(score2d, act2d)


def _conv_finish(p0, p1, si, act, w, bvec, pn, fi):
    def fn(a0, a1, sv, av, wv, bv, pv):
        agg = (a0 + a1)[:, :fi] * sv
        h = (_mm(agg, wv) + bv) * av
        y = _mm(h, pv)
        return h, y

    fo = w.shape[1]
    return _rowcall(fn, NP, 4, [fo, fo], p0, p1, si, act, w, bvec, pn)


def _gate_table(h, y, actn, dgo0, dgo1, dgi0, dgi1, fpad):
    fo = h.shape[1]

    def fn(hv, yv, av, o0, o1, i0, i1):
        so = lax.rsqrt(jnp.maximum(o0 + o1, 1.0))
        si = lax.rsqrt(jnp.maximum(i0 + i1, 1.0))
        t = hv * jax.nn.sigmoid(yv) * (av * so)
        if fpad > fo:
            t = jnp.concatenate(
                [t, jnp.zeros((t.shape[0], fpad - fo), jnp.float32)], axis=1)
        return t, so, si

    return _rowcall(fn, NP, 7, [fpad, 1, 1], h, y, actn, dgo0, dgo1, dgi0, dgi1)


def _up_step(p0, p1, si, act, bvec, skip, so_next, w_next, fi, fpad):
    def fn(a0, a1, sv, av, hsv, sn, bv, wn):
        agg = (a0 + a1)[:, :fi] * sv
        h = (agg + bv) * av
        u = (h + hsv) * sn
        t = _mm(u, wn)
        if fpad > t.shape[1]:
            t = jnp.concatenate(
                [t, jnp.zeros((t.shape[0], fpad - t.shape[1]), jnp.float32)],
                axis=1)
        return t

    return _rowcall(fn, NP, 6, [fpad], p0, p1, si, act, skip, so_next,
                    bvec, w_next)


def _prep_edges(edge_index, e, n, np_rows, ep):
    pad = ep - e
    pidx = (n + (jnp.arange(pad, dtype=jnp.int32) % (np_rows - n))).astype(jnp.int32)
    s2 = jnp.concatenate([edge_index[0], pidx])
    d2 = jnp.concatenate([edge_index[1], pidx])
    return s2, d2


def _to2d(col):
    return col.reshape(col.shape[0] // 128, 128)


def _tocol(arr2d):
    return arr2d.reshape(arr2d.shape[0] * 128, 1)


def kernel(x, params, edge_index, broadcast, edge_index_hr):
    del broadcast
    p = params
    src2, dst2 = _prep_edges(edge_index, E, N, NP, EP)
    hsrc2, hdst2 = _prep_edges(edge_index_hr, EHR, NHR, NHRP, EPH)

    act0 = (jnp.arange(NP) < N).astype(jnp.float32).reshape(NP, 1)
    acthr = (jnp.arange(NHRP) < NHR).astype(jnp.float32).reshape(NHRP, 1)
    xp = jnp.pad(x, ((0, NP - N), (0, 0)))

    def pnorm(mat):
        return mat * lax.rsqrt(jnp.sum(mat * mat))

    ks = [75000, 56250, 42187, 31640]

    d0i = _agg(src2, dst2, act0, NP, 1, 50)
    d0o = _agg(dst2, src2, act0, NP, 1, 50)

    def t1fn(xv, o0, o1, i0, i1):
        so = lax.rsqrt(jnp.maximum(o0 + o1, 1.0))
        si = lax.rsqrt(jnp.maximum(i0 + i1, 1.0))
        return xv * so, so, si

    table1, so0, si0 = _rowcall(t1fn, NP, 5, [1, 1, 1],
                                xp, d0o[0], d0o[1], d0i[0], d0i[1])
    a1 = _agg(src2, dst2, table1, NP, 1, 50)
    h0, y1 = _conv_finish(a1[0], a1[1], si0, act0, p["W_c1"],
                          p["b_c1"].reshape(1, -1), pnorm(p["p1"]), 1)

    hs, ys, acts, sos, sis = [h0], [y1], [act0], [so0], [si0]
    convs = [("c2", 1, 10, 16), ("c3", 10, 14, 16), ("c4", 14, 18, 16),
             ("bn", 18, 18, 18)]
    pools = ["p2", "p3", "p4", None]
    for lvl in range(4):
        actn2d = _topk_mask(_to2d(ys[-1][:, 0:1].reshape(NP)),
                            _to2d(acts[-1].reshape(NP)), ks[lvl])
        actn = _tocol(actn2d)
        dni = _agg(src2, dst2, actn, NP, 1, 50)
        dno = _agg(dst2, src2, actn, NP, 1, 50)
        fo_prev = hs[-1].shape[1]
        fpad = 1 if fo_prev == 1 else (16 if fo_prev <= 16 else 18)
        table, so, si = _gate_table(hs[-1], ys[-1], actn,
                                    dno[0], dno[1], dni[0], dni[1], fpad)
        ag = _agg(src2, dst2, table, NP, fpad, 50)
        name, fi, fo, _ = convs[lvl]
        pool = pools[lvl]
        pmat = pnorm(p[pool]) if pool else jnp.eye(fo, dtype=jnp.float32)
        h, y = _conv_finish(ag[0], ag[1], si, actn, p["W_" + name],
                            p["b_" + name].reshape(1, -1), pmat, fi)
        acts.append(actn)
        sos.append(so)
        sis.append(si)
        hs.append(h)
        ys.append(y)

    def mkfn(fo_t, fpad_t):
        def fn(hbnv, hsv, snv, wnv):
            u = (hbnv + hsv) * snv
            t = _mm(u, wnv)
            if fpad_t > t.shape[1]:
                t = jnp.concatenate(
                    [t, jnp.zeros((t.shape[0], fpad_t - t.shape[1]),
                                  jnp.float32)], axis=1)
            return t
        return fn

    tab = _rowcall(mkfn(14, 16), NP, 3, [16], hs[4], hs[3], sos[3], p["W_u1"])
    ag = _agg(src2, dst2, tab, NP, 16, 50)
    tab = _up_step(ag[0], ag[1], sis[3], acts[3], p["b_u1"].reshape(1, -1),
                   hs[2], sos[2], p["W_u2"], 14, 16)
    ag = _agg(src2, dst2, tab, NP, 16, 50)
    tab = _up_step(ag[0], ag[1], sis[2], acts[2], p["b_u2"].reshape(1, -1),
                   hs[1], sos[1], p["W_u3"], 10, 1)
    ag = _agg(src2, dst2, tab, NP, 1, 50)
    tab = _up_step(ag[0], ag[1], sis[1], acts[1], p["b_u3"].reshape(1, -1),
                   hs[0], sos[0], p["W_u4"], 1, 1)
    ag = _agg(src2, dst2, tab, NP, 1, 50)

    def u4fn(a0, a1, sv, av, bv):
        return (a0 + a1) * sv * av + bv * av

    u4 = _rowcall(u4fn, NP, 4, [1], ag[0], ag[1], si0, act0,
                  p["b_u4"].reshape(1, -1))

    neu = jnp.concatenate([u4[:N], u4[:N]], axis=1).reshape(2 * N, 1)
    neu = jnp.pad(neu, ((0, NHRP - NHR), (0, 0)))

    dhi = _agg(hsrc2, hdst2, acthr, NHRP, 1, 98)
    dho = _agg(hdst2, hsrc2, acthr, NHRP, 1, 98)

    def thrfn(nv, o0, o1, i0, i1):
        so = lax.rsqrt(jnp.maximum(o0 + o1, 1.0))
        si = lax.rsqrt(jnp.maximum(i0 + i1, 1.0))
        return nv * so, si

    tabbc, sihr = _rowcall(thrfn, NHRP, 5, [1, 1],
                           neu, dho[0], dho[1], dhi[0], dhi[1])
    abc = _agg(hsrc2, hdst2, tabbc, NHRP, 1, 98)

    projs = [(p["projW%d" % j], p["projb%d" % j].reshape(1, -1))
             for j in range(6)]

    def headfn(a0, a1, sv, av, wbc, bbc, *wb):
        h = (a0 + a1) * sv * wbc + bbc
        h = h * av
        for j in range(6):
            h = _mm(h, wb[2 * j]) + wb[2 * j + 1]
            if j < 5:
                h = jnp.where(h >= 0, h, -0.8 * h)
                h = jnp.tanh(h)
        return h

    consts = [p["W_bc"], p["b_bc"].reshape(1, -1)]
    for w, b in projs:
        consts.extend([w, b])
    out = _rowcall(headfn, NHRP, 4, [3], abc[0], abc[1], sihr, acthr, *consts)
    return out[:NHR]


# speedup vs baseline: 43.2524x; 1.2185x over previous
import functools

import jax
import jax.numpy as jnp
from jax import lax
from jax.experimental import pallas as pl
from jax.experimental.pallas import tpu as pltpu
from jax.experimental.pallas import tpu_sc as plsc

N = 100000
NP = 100352
NHR = 200000
NHRP = 200704
E = 1600000
EHR = 3200000
EP = 1638400
EPH = 3211264
NC, NS = 2, 16
CH = 1024
ZR = 784
BR = 1024

_I32_MIN = -2147483648


@functools.lru_cache(None)
def _edge_agg(np_rows, f, nblocks):
    rows_per_tile = np_rows // NS
    nz = rows_per_tile // ZR
    mesh = plsc.VectorSubcoreMesh(
        core_axis_name="c", subcore_axis_name="s", num_cores=NC, num_subcores=NS
    )

    one_d = f == 1

    def body(src_hbm, dst_hbm, table_hbm, zrow_hbm, out_hbm,
             acc, tab_sh, idx_v, rows_v, gsem, ssem, isem):
        c = lax.axis_index("c")
        s = lax.axis_index("s")
        wid = c * NS + s
        for i in range(nz):
            if one_d:
                pltpu.sync_copy(zrow_hbm,
                                acc.at[pl.ds(s * rows_per_tile + i * ZR, ZR)])
            else:
                pltpu.sync_copy(zrow_hbm,
                                acc.at[pl.ds(s * rows_per_tile + i * ZR, ZR), :])
        if one_d:
            pltpu.sync_copy(table_hbm.at[pl.ds(s * rows_per_tile, rows_per_tile)],
                            tab_sh.at[pl.ds(s * rows_per_tile, rows_per_tile)])
        plsc.subcore_barrier()
        gather_src = tab_sh if one_d else table_hbm

        base = wid * nblocks * CH

        def start_idx(b, p):
            pltpu.async_copy(src_hbm.at[pl.ds(base + b * CH, CH)],
                             idx_v.at[p, 0], isem.at[p, 0])
            pltpu.async_copy(dst_hbm.at[pl.ds(base + b * CH, CH)],
                             idx_v.at[p, 1], isem.at[p, 1])

        def wait_idx(p):
            pltpu.make_async_copy(src_hbm.at[pl.ds(0, CH)], idx_v.at[p, 0],
                                  isem.at[p, 0]).wait()
            pltpu.make_async_copy(dst_hbm.at[pl.ds(0, CH)], idx_v.at[p, 1],
                                  isem.at[p, 1]).wait()

        def do_block(b, p):
            wait_idx(p)

            @pl.when(b + 1 < nblocks)
            def _():
                start_idx(b + 1, 1 - p)

            pltpu.async_copy(gather_src.at[idx_v.at[p, 0]], rows_v, gsem).wait()
            pltpu.async_copy(rows_v, acc.at[idx_v.at[p, 1]], ssem,
                             add=True).wait()

        start_idx(0, 0)

        def loop_body(i, carry):
            b = i * 2
            do_block(b, 0)

            @pl.when(b + 1 < nblocks)
            def _():
                do_block(b + 1, 1)

            return carry

        lax.fori_loop(0, (nblocks + 1) // 2, loop_body, jnp.int32(0))

        plsc.subcore_barrier()
        if one_d:
            pltpu.sync_copy(acc.at[pl.ds(s * rows_per_tile, rows_per_tile)],
                            out_hbm.at[c, pl.ds(s * rows_per_tile, rows_per_tile)])
        else:
            pltpu.sync_copy(acc.at[pl.ds(s * rows_per_tile, rows_per_tile), :],
                            out_hbm.at[c, pl.ds(s * rows_per_tile, rows_per_tile), :])

    out_sh = (NC, np_rows) if one_d else (NC, np_rows, f)
    acc_sh = (np_rows,) if one_d else (np_rows, f)
    rows_sh = (CH,) if one_d else (CH, f)
    return pl.kernel(
        body,
        out_type=jax.ShapeDtypeStruct(out_sh, jnp.float32),
        mesh=mesh,
        compiler_params=pltpu.CompilerParams(use_tc_tiling_on_sc=False),
        scratch_types=[
            pltpu.VMEM_SHARED(acc_sh, jnp.float32),
            pltpu.VMEM_SHARED((np_rows,) if one_d else (8,), jnp.float32),
            pltpu.VMEM((2, 2, CH), jnp.int32),
            pltpu.VMEM(rows_sh, jnp.float32),
            pltpu.SemaphoreType.DMA,
            pltpu.SemaphoreType.DMA,
            pltpu.SemaphoreType.DMA((2, 2)),
        ],
    )


@functools.lru_cache(None)
def _deg_kernel(np_rows, nblocks):
    rows_per_tile = np_rows // NS
    nz = rows_per_tile // ZR
    mesh = plsc.VectorSubcoreMesh(
        core_axis_name="c", subcore_axis_name="s", num_cores=NC, num_subcores=NS
    )

    def body(src_hbm, dst_hbm, act_hbm, zrow_hbm, out_hbm,
             acc_o, acc_i, tab_sh, idx_v, rows_v, gsem, ssem, isem):
        c = lax.axis_index("c")
        s = lax.axis_index("s")
        wid = c * NS + s
        for i in range(nz):
            pltpu.sync_copy(zrow_hbm,
                            acc_o.at[pl.ds(s * rows_per_tile + i * ZR, ZR)])
            pltpu.sync_copy(zrow_hbm,
                            acc_i.at[pl.ds(s * rows_per_tile + i * ZR, ZR)])
        pltpu.sync_copy(act_hbm.at[pl.ds(s * rows_per_tile, rows_per_tile)],
                        tab_sh.at[pl.ds(s * rows_per_tile, rows_per_tile)])
        plsc.subcore_barrier()

        base = wid * nblocks * CH

        def start_idx(b, p):
            pltpu.async_copy(src_hbm.at[pl.ds(base + b * CH, CH)],
                             idx_v.at[p, 0], isem.at[p, 0])
            pltpu.async_copy(dst_hbm.at[pl.ds(base + b * CH, CH)],
                             idx_v.at[p, 1], isem.at[p, 1])

        def wait_idx(p):
            pltpu.make_async_copy(src_hbm.at[pl.ds(0, CH)], idx_v.at[p, 0],
                                  isem.at[p, 0]).wait()
            pltpu.make_async_copy(dst_hbm.at[pl.ds(0, CH)], idx_v.at[p, 1],
                                  isem.at[p, 1]).wait()

        def do_block(b, p):
            wait_idx(p)

            @pl.when(b + 1 < nblocks)
            def _():
                start_idx(b + 1, 1 - p)

            g0 = pltpu.async_copy(tab_sh.at[idx_v.at[p, 0]], rows_v.at[0],
                                  gsem)
            g1 = pltpu.async_copy(tab_sh.at[idx_v.at[p, 1]], rows_v.at[1],
                                  gsem)
            g0.wait()
            g1.wait()
            s0 = pltpu.async_copy(rows_v.at[1], acc_o.at[idx_v.at[p, 0]],
                                  ssem, add=True)
            s1 = pltpu.async_copy(rows_v.at[0], acc_i.at[idx_v.at[p, 1]],
                                  ssem, add=True)
            s0.wait()
            s1.wait()

        start_idx(0, 0)

        def loop_body(i, carry):
            b = i * 2
            do_block(b, 0)

            @pl.when(b + 1 < nblocks)
            def _():
                do_block(b + 1, 1)

            return carry

        lax.fori_loop(0, (nblocks + 1) // 2, loop_body, jnp.int32(0))

        plsc.subcore_barrier()
        pltpu.sync_copy(acc_o.at[pl.ds(s * rows_per_tile, rows_per_tile)],
                        out_hbm.at[c, 0, pl.ds(s * rows_per_tile, rows_per_tile)])
        pltpu.sync_copy(acc_i.at[pl.ds(s * rows_per_tile, rows_per_tile)],
                        out_hbm.at[c, 1, pl.ds(s * rows_per_tile, rows_per_tile)])

    return pl.kernel(
        body,
        out_type=jax.ShapeDtypeStruct((NC, 2, np_rows), jnp.float32),
        mesh=mesh,
        compiler_params=pltpu.CompilerParams(use_tc_tiling_on_sc=False),
        scratch_types=[
            pltpu.VMEM_SHARED((np_rows,), jnp.float32),
            pltpu.VMEM_SHARED((np_rows,), jnp.float32),
            pltpu.VMEM_SHARED((np_rows,), jnp.float32),
            pltpu.VMEM((2, 2, CH), jnp.int32),
            pltpu.VMEM((2, CH), jnp.float32),
            pltpu.SemaphoreType.DMA,
            pltpu.SemaphoreType.DMA,
            pltpu.SemaphoreType.DMA((2, 2)),
        ],
    )


def _deg(src, dst, act, np_rows, nblocks):
    z = jnp.zeros((ZR,), jnp.float32)
    out = _deg_kernel(np_rows, nblocks)(src, dst, act.reshape(np_rows), z)
    return (out[0, 0].reshape(np_rows, 1), out[1, 0].reshape(np_rows, 1),
            out[0, 1].reshape(np_rows, 1), out[1, 1].reshape(np_rows, 1))


def _agg(src2d, dst2d, table, np_rows, f, nblocks):
    if f > 16:
        parts = [_agg(src2d, dst2d, table[:, :16], np_rows, 16, nblocks)]
        for j in range(16, f):
            parts.append(_agg(src2d, dst2d, table[:, j:j + 1], np_rows, 1,
                              nblocks))
        return jnp.concatenate(parts, axis=2)
    if f == 1:
        z = jnp.zeros((ZR,), jnp.float32)
        out = _edge_agg(np_rows, 1, nblocks)(src2d, dst2d,
                                             table.reshape(np_rows), z)
        return out.reshape(NC, np_rows, 1)
    z = jnp.zeros((ZR, f), jnp.float32)
    return _edge_agg(np_rows, f, nblocks)(src2d, dst2d, table, z)


def _mm(x, w):
    if w.shape[0] == 1:
        return x * w
    return jnp.dot(x, w, preferred_element_type=jnp.float32)


def _rowcall(fn, np_rows, n_row_args, out_widths, *args):
    row_args = args[:n_row_args]
    const_args = args[n_row_args:]
    grid = np_rows // BR
    in_specs = (
        [pl.BlockSpec((BR, a.shape[1]), lambda i: (i, 0)) for a in row_args]
        + [pl.BlockSpec(c.shape, lambda i: (0, 0)) for c in const_args]
    )
    out_specs = [pl.BlockSpec((BR, w), lambda i: (i, 0)) for w in out_widths]
    out_shape = [jax.ShapeDtypeStruct((np_rows, w), jnp.float32) for w in out_widths]

    def kfn(*refs):
        nin = len(args)
        vals = [r[...] for r in refs[:nin]]
        outs = fn(*vals)
        if len(out_widths) == 1:
            outs = (outs,)
        for r, o in zip(refs[nin:], outs):
            r[...] = o

    res = pl.pallas_call(kfn, grid=(grid,), in_specs=in_specs,
                         out_specs=out_specs, out_shape=out_shape)(*args)
    return res[0] if len(out_widths) == 1 else res


def _topk_mask(score2d, act2d, kk):
    rows = score2d.shape[0]

    def kfn(s_ref, a_ref, o_ref):
        sc = s_ref[...]
        a = a_ref[...]
        b = lax.bitcast_convert_type(sc, jnp.int32)
        key = jnp.where(b < 0, jnp.bitwise_xor(b, jnp.int32(0x7FFFFFFF)), b)
        key = jnp.where(a > 0, key, _I32_MIN)

        def step(i, cu):
            bit = jnp.left_shift(jnp.int32(1), 31 - i)
            c2 = jnp.bitwise_or(cu, bit)
            thr = jnp.bitwise_xor(c2, _I32_MIN)
            cnt = jnp.sum((key >= thr).astype(jnp.int32))
            return jnp.where(cnt >= kk, c2, cu)

        cu = lax.fori_loop(0, 32, step, jnp.int32(0))
        thr = jnp.bitwise_xor(cu, _I32_MIN)
        o_ref[...] = ((key >= thr) & (a > 0)).astype(jnp.float32)

    return pl.pallas_call(
        kfn,
        out_shape=jax.ShapeDtypeStruct((rows, 128), jnp.float32),
    )(score2d, act2d)


def _conv_finish(p0, p1, si, act, w, bvec, pn, fi):
    def fn(a0, a1, sv, av, wv, bv, pv):
        agg = (a0 + a1)[:, :fi] * sv
        h = (_mm(agg, wv) + bv) * av
        y = _mm(h, pv)
        return h, y

    fo = w.shape[1]
    return _rowcall(fn, NP, 4, [fo, fo], p0, p1, si, act, w, bvec, pn)


def _gate_table(h, y, actn, dgo0, dgo1, dgi0, dgi1, fpad):
    fo = h.shape[1]

    def fn(hv, yv, av, o0, o1, i0, i1):
        so = lax.rsqrt(jnp.maximum(o0 + o1, 1.0))
        si = lax.rsqrt(jnp.maximum(i0 + i1, 1.0))
        t = hv * jax.nn.sigmoid(yv) * (av * so)
        if fpad > fo:
            t = jnp.concatenate(
                [t, jnp.zeros((t.shape[0], fpad - fo), jnp.float32)], axis=1)
        return t, so, si

    return _rowcall(fn, NP, 7, [fpad, 1, 1], h, y, actn, dgo0, dgo1, dgi0, dgi1)


def _up_step(p0, p1, si, act, bvec, skip, so_next, w_next, fi, fpad):
    def fn(a0, a1, sv, av, hsv, sn, bv, wn):
        agg = (a0 + a1)[:, :fi] * sv
        h = (agg + bv) * av
        u = (h + hsv) * sn
        t = _mm(u, wn)
        if fpad > t.shape[1]:
            t = jnp.concatenate(
                [t, jnp.zeros((t.shape[0], fpad - t.shape[1]), jnp.float32)],
                axis=1)
        return t

    return _rowcall(fn, NP, 6, [fpad], p0, p1, si, act, skip, so_next,
                    bvec, w_next)


def _prep_edges(edge_index, e, n, np_rows, ep):
    pad = ep - e
    pidx = (n + (jnp.arange(pad, dtype=jnp.int32) % (np_rows - n))).astype(jnp.int32)
    s2 = jnp.concatenate([edge_index[0], pidx])
    d2 = jnp.concatenate([edge_index[1], pidx])
    return s2, d2


def _to2d(col):
    return col.reshape(col.shape[0] // 128, 128)


def _tocol(arr2d):
    return arr2d.reshape(arr2d.shape[0] * 128, 1)


def kernel(x, params, edge_index, broadcast, edge_index_hr):
    del broadcast
    p = params
    src2, dst2 = _prep_edges(edge_index, E, N, NP, EP)
    hsrc2, hdst2 = _prep_edges(edge_index_hr, EHR, NHR, NHRP, EPH)

    act0 = (jnp.arange(NP) < N).astype(jnp.float32).reshape(NP, 1)
    acthr = (jnp.arange(NHRP) < NHR).astype(jnp.float32).reshape(NHRP, 1)
    xp = jnp.pad(x, ((0, NP - N), (0, 0)))

    def pnorm(mat):
        return mat * lax.rsqrt(jnp.sum(mat * mat))

    ks = [75000, 56250, 42187, 31640]

    dgo0, dgo1, dgi0, dgi1 = _deg(src2, dst2, act0, NP, 50)

    def t1fn(xv, o0, o1, i0, i1):
        so = lax.rsqrt(jnp.maximum(o0 + o1, 1.0))
        si = lax.rsqrt(jnp.maximum(i0 + i1, 1.0))
        return xv * so, so, si

    table1, so0, si0 = _rowcall(t1fn, NP, 5, [1, 1, 1],
                                xp, dgo0, dgo1, dgi0, dgi1)
    a1 = _agg(src2, dst2, table1, NP, 1, 50)
    h0, y1 = _conv_finish(a1[0], a1[1], si0, act0, p["W_c1"],
                          p["b_c1"].reshape(1, -1), pnorm(p["p1"]), 1)

    hs, ys, acts, sos, sis = [h0], [y1], [act0], [so0], [si0]
    convs = [("c2", 1, 10, 16), ("c3", 10, 14, 16), ("c4", 14, 18, 16),
             ("bn", 18, 18, 18)]
    pools = ["p2", "p3", "p4", None]
    for lvl in range(4):
        actn2d = _topk_mask(_to2d(ys[-1][:, 0:1].reshape(NP)),
                            _to2d(acts[-1].reshape(NP)), ks[lvl])
        actn = _tocol(actn2d)
        no0, no1, ni0, ni1 = _deg(src2, dst2, actn, NP, 50)
        fo_prev = hs[-1].shape[1]
        fpad = 1 if fo_prev == 1 else (16 if fo_prev <= 16 else 18)
        table, so, si = _gate_table(hs[-1], ys[-1], actn,
                                    no0, no1, ni0, ni1, fpad)
        ag = _agg(src2, dst2, table, NP, fpad, 50)
        name, fi, fo, _ = convs[lvl]
        pool = pools[lvl]
        pmat = pnorm(p[pool]) if pool else jnp.eye(fo, dtype=jnp.float32)
        h, y = _conv_finish(ag[0], ag[1], si, actn, p["W_" + name],
                            p["b_" + name].reshape(1, -1), pmat, fi)
        acts.append(actn)
        sos.append(so)
        sis.append(si)
        hs.append(h)
        ys.append(y)

    def mkfn(fo_t, fpad_t):
        def fn(hbnv, hsv, snv, wnv):
            u = (hbnv + hsv) * snv
            t = _mm(u, wnv)
            if fpad_t > t.shape[1]:
                t = jnp.concatenate(
                    [t, jnp.zeros((t.shape[0], fpad_t - t.shape[1]),
                                  jnp.float32)], axis=1)
            return t
        return fn

    tab = _rowcall(mkfn(14, 16), NP, 3, [16], hs[4], hs[3], sos[3], p["W_u1"])
    ag = _agg(src2, dst2, tab, NP, 16, 50)
    tab = _up_step(ag[0], ag[1], sis[3], acts[3], p["b_u1"].reshape(1, -1),
                   hs[2], sos[2], p["W_u2"], 14, 16)
    ag = _agg(src2, dst2, tab, NP, 16, 50)
    tab = _up_step(ag[0], ag[1], sis[2], acts[2], p["b_u2"].reshape(1, -1),
                   hs[1], sos[1], p["W_u3"], 10, 1)
    ag = _agg(src2, dst2, tab, NP, 1, 50)
    tab = _up_step(ag[0], ag[1], sis[1], acts[1], p["b_u3"].reshape(1, -1),
                   hs[0], sos[0], p["W_u4"], 1, 1)
    ag = _agg(src2, dst2, tab, NP, 1, 50)

    def u4fn(a0, a1, sv, av, bv):
        return (a0 + a1) * sv * av + bv * av

    u4 = _rowcall(u4fn, NP, 4, [1], ag[0], ag[1], si0, act0,
                  p["b_u4"].reshape(1, -1))

    neu = jnp.concatenate([u4[:N], u4[:N]], axis=1).reshape(2 * N, 1)
    neu = jnp.pad(neu, ((0, NHRP - NHR), (0, 0)))

    ho0, ho1, hi0, hi1 = _deg(hsrc2, hdst2, acthr, NHRP, 98)

    def thrfn(nv, o0, o1, i0, i1):
        so = lax.rsqrt(jnp.maximum(o0 + o1, 1.0))
        si = lax.rsqrt(jnp.maximum(i0 + i1, 1.0))
        return nv * so, si

    tabbc, sihr = _rowcall(thrfn, NHRP, 5, [1, 1],
                           neu, ho0, ho1, hi0, hi1)
    abc = _agg(hsrc2, hdst2, tabbc, NHRP, 1, 98)

    projs = [(p["projW%d" % j], p["projb%d" % j].reshape(1, -1))
             for j in range(6)]

    def headfn(a0, a1, sv, av, wbc, bbc, *wb):
        h = (a0 + a1) * sv * wbc + bbc
        h = h * av
        for j in range(6):
            h = _mm(h, wb[2 * j]) + wb[2 * j + 1]
            if j < 5:
                h = jnp.where(h >= 0, h, -0.8 * h)
                h = jnp.tanh(h)
        return h

    consts = [p["W_bc"], p["b_bc"].reshape(1, -1)]
    for w, b in projs:
        consts.extend([w, b])
    out = _rowcall(headfn, NHRP, 4, [3], abc[0], abc[1], sihr, acthr, *consts)
    return out[:NHR]


# speedup vs baseline: 48.7105x; 1.1262x over previous
import functools

import jax
import jax.numpy as jnp
from jax import lax
from jax.experimental import pallas as pl
from jax.experimental.pallas import tpu as pltpu
from jax.experimental.pallas import tpu_sc as plsc

N = 100000
NP = 100352
NHR = 200000
NHRP = 200704
E = 1600000
EHR = 3200000
EP = 1638400
EPH = 3211264
NC, NS = 2, 16
CH = 1024
ZR = 784
BR = 3136

_I32_MIN = -2147483648


@functools.lru_cache(None)
def _edge_agg(np_rows, f, nblocks):
    rows_per_tile = np_rows // NS
    nz = rows_per_tile // ZR
    mesh = plsc.VectorSubcoreMesh(
        core_axis_name="c", subcore_axis_name="s", num_cores=NC, num_subcores=NS
    )

    one_d = f == 1

    def body(src_hbm, dst_hbm, table_hbm, zrow_hbm, out_hbm,
             acc, tab_sh, idx_v, rows_v, gsem, ssem, ssem2, isem):
        c = lax.axis_index("c")
        s = lax.axis_index("s")
        wid = c * NS + s
        for i in range(nz):
            if one_d:
                pltpu.sync_copy(zrow_hbm,
                                acc.at[pl.ds(s * rows_per_tile + i * ZR, ZR)])
            else:
                pltpu.sync_copy(zrow_hbm,
                                acc.at[pl.ds(s * rows_per_tile + i * ZR, ZR), :])
        if one_d:
            pltpu.sync_copy(table_hbm.at[pl.ds(s * rows_per_tile, rows_per_tile)],
                            tab_sh.at[pl.ds(s * rows_per_tile, rows_per_tile)])
        plsc.subcore_barrier()
        gather_src = tab_sh if one_d else table_hbm

        base = wid * nblocks * CH

        def start_idx(b, q):
            pltpu.async_copy(src_hbm.at[pl.ds(base + b * CH, CH)],
                             idx_v.at[q, 0], isem.at[q, 0])
            pltpu.async_copy(dst_hbm.at[pl.ds(base + b * CH, CH)],
                             idx_v.at[q, 1], isem.at[q, 1])

        def wait_idx(q):
            pltpu.make_async_copy(src_hbm.at[pl.ds(0, CH)], idx_v.at[q, 0],
                                  isem.at[q, 0]).wait()
            pltpu.make_async_copy(dst_hbm.at[pl.ds(0, CH)], idx_v.at[q, 1],
                                  isem.at[q, 1]).wait()

        def wait_scatter(q):
            pltpu.make_async_copy(rows_v.at[q], acc.at[idx_v.at[q, 1]],
                                  ssem2.at[q]).wait()

        if one_d:
            def do_block(b, q):
                wait_idx(q)
                qn = (q + 1) % 3

                @pl.when(b >= 2)
                def _():
                    wait_scatter(qn)

                @pl.when(b + 1 < nblocks)
                def _():
                    start_idx(b + 1, qn)

                pltpu.async_copy(gather_src.at[idx_v.at[q, 0]], rows_v.at[q],
                                 gsem).wait()
                pltpu.async_copy(rows_v.at[q], acc.at[idx_v.at[q, 1]],
                                 ssem2.at[q], add=True)

            start_idx(0, 0)

            def loop_body(i, carry):
                b = i * 3
                do_block(b, 0)

                @pl.when(b + 1 < nblocks)
                def _():
                    do_block(b + 1, 1)

                @pl.when(b + 2 < nblocks)
                def _():
                    do_block(b + 2, 2)

                return carry

            lax.fori_loop(0, (nblocks + 2) // 3, loop_body, jnp.int32(0))
            if nblocks >= 2:
                wait_scatter((nblocks - 2) % 3)
            wait_scatter((nblocks - 1) % 3)
        else:
            def do_block(b, p):
                wait_idx(p)

                @pl.when(b + 1 < nblocks)
                def _():
                    start_idx(b + 1, 1 - p)

                pltpu.async_copy(gather_src.at[idx_v.at[p, 0]], rows_v,
                                 gsem).wait()
                pltpu.async_copy(rows_v, acc.at[idx_v.at[p, 1]], ssem,
                                 add=True).wait()

            start_idx(0, 0)

            def loop_body(i, carry):
                b = i * 2
                do_block(b, 0)

                @pl.when(b + 1 < nblocks)
                def _():
                    do_block(b + 1, 1)

                return carry

            lax.fori_loop(0, (nblocks + 1) // 2, loop_body, jnp.int32(0))

        plsc.subcore_barrier()
        if one_d:
            pltpu.sync_copy(acc.at[pl.ds(s * rows_per_tile, rows_per_tile)],
                            out_hbm.at[c, pl.ds(s * rows_per_tile, rows_per_tile)])
        else:
            pltpu.sync_copy(acc.at[pl.ds(s * rows_per_tile, rows_per_tile), :],
                            out_hbm.at[c, pl.ds(s * rows_per_tile, rows_per_tile), :])

    out_sh = (NC, np_rows) if one_d else (NC, np_rows, f)
    acc_sh = (np_rows,) if one_d else (np_rows, f)
    rows_sh = (3, CH) if one_d else (CH, f)
    idx_sh = (3, 2, CH) if one_d else (2, 2, CH)
    isem_sh = (3, 2) if one_d else (2, 2)
    return pl.kernel(
        body,
        out_type=jax.ShapeDtypeStruct(out_sh, jnp.float32),
        mesh=mesh,
        compiler_params=pltpu.CompilerParams(use_tc_tiling_on_sc=False),
        scratch_types=[
            pltpu.VMEM_SHARED(acc_sh, jnp.float32),
            pltpu.VMEM_SHARED((np_rows,) if one_d else (8,), jnp.float32),
            pltpu.VMEM(idx_sh, jnp.int32),
            pltpu.VMEM(rows_sh, jnp.float32),
            pltpu.SemaphoreType.DMA,
            pltpu.SemaphoreType.DMA,
            pltpu.SemaphoreType.DMA((3,)),
            pltpu.SemaphoreType.DMA(isem_sh),
        ],
    )


@functools.lru_cache(None)
def _deg_kernel(np_rows, nblocks):
    rows_per_tile = np_rows // NS
    nz = rows_per_tile // ZR
    mesh = plsc.VectorSubcoreMesh(
        core_axis_name="c", subcore_axis_name="s", num_cores=NC, num_subcores=NS
    )

    def body(src_hbm, dst_hbm, act_hbm, zrow_hbm, out_hbm,
             acc_o, acc_i, tab_sh, idx_v, rows_v, gsem, ssem2, isem):
        c = lax.axis_index("c")
        s = lax.axis_index("s")
        wid = c * NS + s
        for i in range(nz):
            pltpu.sync_copy(zrow_hbm,
                            acc_o.at[pl.ds(s * rows_per_tile + i * ZR, ZR)])
            pltpu.sync_copy(zrow_hbm,
                            acc_i.at[pl.ds(s * rows_per_tile + i * ZR, ZR)])
        pltpu.sync_copy(act_hbm.at[pl.ds(s * rows_per_tile, rows_per_tile)],
                        tab_sh.at[pl.ds(s * rows_per_tile, rows_per_tile)])
        plsc.subcore_barrier()

        base = wid * nblocks * CH

        def start_idx(b, q):
            pltpu.async_copy(src_hbm.at[pl.ds(base + b * CH, CH)],
                             idx_v.at[q, 0], isem.at[q, 0])
            pltpu.async_copy(dst_hbm.at[pl.ds(base + b * CH, CH)],
                             idx_v.at[q, 1], isem.at[q, 1])

        def wait_idx(q):
            pltpu.make_async_copy(src_hbm.at[pl.ds(0, CH)], idx_v.at[q, 0],
                                  isem.at[q, 0]).wait()
            pltpu.make_async_copy(dst_hbm.at[pl.ds(0, CH)], idx_v.at[q, 1],
                                  isem.at[q, 1]).wait()

        def wait_scatter(q):
            pltpu.make_async_copy(rows_v.at[q, 1], acc_o.at[idx_v.at[q, 0]],
                                  ssem2.at[q]).wait()
            pltpu.make_async_copy(rows_v.at[q, 0], acc_i.at[idx_v.at[q, 1]],
                                  ssem2.at[q]).wait()

        def do_block(b, q):
            wait_idx(q)
            qn = (q + 1) % 3

            @pl.when(b >= 2)
            def _():
                wait_scatter(qn)

            @pl.when(b + 1 < nblocks)
            def _():
                start_idx(b + 1, qn)

            g0 = pltpu.async_copy(tab_sh.at[idx_v.at[q, 0]], rows_v.at[q, 0],
                                  gsem)
            g1 = pltpu.async_copy(tab_sh.at[idx_v.at[q, 1]], rows_v.at[q, 1],
                                  gsem)
            g0.wait()
            g1.wait()
            pltpu.async_copy(rows_v.at[q, 1], acc_o.at[idx_v.at[q, 0]],
                             ssem2.at[q], add=True)
            pltpu.async_copy(rows_v.at[q, 0], acc_i.at[idx_v.at[q, 1]],
                             ssem2.at[q], add=True)

        start_idx(0, 0)

        def loop_body(i, carry):
            b = i * 3
            do_block(b, 0)

            @pl.when(b + 1 < nblocks)
            def _():
                do_block(b + 1, 1)

            @pl.when(b + 2 < nblocks)
            def _():
                do_block(b + 2, 2)

            return carry

        lax.fori_loop(0, (nblocks + 2) // 3, loop_body, jnp.int32(0))
        if nblocks >= 2:
            wait_scatter((nblocks - 2) % 3)
        wait_scatter((nblocks - 1) % 3)

        plsc.subcore_barrier()
        pltpu.sync_copy(acc_o.at[pl.ds(s * rows_per_tile, rows_per_tile)],
                        out_hbm.at[c, 0, pl.ds(s * rows_per_tile, rows_per_tile)])
        pltpu.sync_copy(acc_i.at[pl.ds(s * rows_per_tile, rows_per_tile)],
                        out_hbm.at[c, 1, pl.ds(s * rows_per_tile, rows_per_tile)])

    return pl.kernel(
        body,
        out_type=jax.ShapeDtypeStruct((NC, 2, np_rows), jnp.float32),
        mesh=mesh,
        compiler_params=pltpu.CompilerParams(use_tc_tiling_on_sc=False),
        scratch_types=[
            pltpu.VMEM_SHARED((np_rows,), jnp.float32),
            pltpu.VMEM_SHARED((np_rows,), jnp.float32),
            pltpu.VMEM_SHARED((np_rows,), jnp.float32),
            pltpu.VMEM((3, 2, CH), jnp.int32),
            pltpu.VMEM((3, 2, CH), jnp.float32),
            pltpu.SemaphoreType.DMA,
            pltpu.SemaphoreType.DMA((3,)),
            pltpu.SemaphoreType.DMA((3, 2)),
        ],
    )


def _deg(src, dst, act, np_rows, nblocks):
    z = jnp.zeros((ZR,), jnp.float32)
    out = _deg_kernel(np_rows, nblocks)(src, dst, act.reshape(np_rows), z)
    return (out[0, 0].reshape(np_rows, 1), out[1, 0].reshape(np_rows, 1),
            out[0, 1].reshape(np_rows, 1), out[1, 1].reshape(np_rows, 1))


def _agg(src2d, dst2d, table, np_rows, f, nblocks):
    if f > 16:
        parts = [_agg(src2d, dst2d, table[:, :16], np_rows, 16, nblocks)]
        for j in range(16, f):
            parts.append(_agg(src2d, dst2d, table[:, j:j + 1], np_rows, 1,
                              nblocks))
        return jnp.concatenate(parts, axis=2)
    if f == 1:
        z = jnp.zeros((ZR,), jnp.float32)
        out = _edge_agg(np_rows, 1, nblocks)(src2d, dst2d,
                                             table.reshape(np_rows), z)
        return out.reshape(NC, np_rows, 1)
    z = jnp.zeros((ZR, f), jnp.float32)
    return _edge_agg(np_rows, f, nblocks)(src2d, dst2d, table, z)


def _mm(x, w):
    if w.shape[0] == 1:
        return x * w
    return jnp.dot(x, w, preferred_element_type=jnp.float32)


def _rowcall(fn, np_rows, n_row_args, out_widths, *args):
    row_args = args[:n_row_args]
    const_args = args[n_row_args:]
    grid = np_rows // BR
    in_specs = (
        [pl.BlockSpec((BR, a.shape[1]), lambda i: (i, 0)) for a in row_args]
        + [pl.BlockSpec(c.shape, lambda i: (0, 0)) for c in const_args]
    )
    out_specs = [pl.BlockSpec((BR, w), lambda i: (i, 0)) for w in out_widths]
    out_shape = [jax.ShapeDtypeStruct((np_rows, w), jnp.float32) for w in out_widths]

    def kfn(*refs):
        nin = len(args)
        vals = [r[...] for r in refs[:nin]]
        outs = fn(*vals)
        if len(out_widths) == 1:
            outs = (outs,)
        for r, o in zip(refs[nin:], outs):
            r[...] = o

    res = pl.pallas_call(kfn, grid=(grid,), in_specs=in_specs,
                         out_specs=out_specs, out_shape=out_shape)(*args)
    return res[0] if len(out_widths) == 1 else res


def _topk_mask(score2d, act2d, kk):
    rows = score2d.shape[0]

    def kfn(s_ref, a_ref, o_ref):
        sc = s_ref[...]
        a = a_ref[...]
        b = lax.bitcast_convert_type(sc, jnp.int32)
        key = jnp.where(b < 0, jnp.bitwise_xor(b, jnp.int32(0x7FFFFFFF)), b)
        key = jnp.where(a > 0, key, _I32_MIN)

        def step(i, cu):
            bit = jnp.left_shift(jnp.int32(1), 31 - i)
            c2 = jnp.bitwise_or(cu, bit)
            thr = jnp.bitwise_xor(c2, _I32_MIN)
            cnt = jnp.sum((key >= thr).astype(jnp.int32))
            return jnp.where(cnt >= kk, c2, cu)

        cu = lax.fori_loop(0, 32, step, jnp.int32(0))
        thr = jnp.bitwise_xor(cu, _I32_MIN)
        o_ref[...] = ((key >= thr) & (a > 0)).astype(jnp.float32)

    return pl.pallas_call(
        kfn,
        out_shape=jax.ShapeDtypeStruct((rows, 128), jnp.float32),
    )(score2d, act2d)


def _conv_finish(p0, p1, si, act, w, bvec, pn, fi):
    def fn(a0, a1, sv, av, wv, bv, pv):
        agg = (a0 + a1)[:, :fi] * sv
        h = (_mm(agg, wv) + bv) * av
        y = _mm(h, pv)
        return h, y

    fo = w.shape[1]
    return _rowcall(fn, NP, 4, [fo, fo], p0, p1, si, act, w, bvec, pn)


def _gate_table(h, y, actn, dgo0, dgo1, dgi0, dgi1, fpad):
    fo = h.shape[1]

    def fn(hv, yv, av, o0, o1, i0, i1):
        so = lax.rsqrt(jnp.maximum(o0 + o1, 1.0))
        si = lax.rsqrt(jnp.maximum(i0 + i1, 1.0))
        t = hv * jax.nn.sigmoid(yv) * (av * so)
        if fpad > fo:
            t = jnp.concatenate(
                [t, jnp.zeros((t.shape[0], fpad - fo), jnp.float32)], axis=1)
        return t, so, si

    return _rowcall(fn, NP, 7, [fpad, 1, 1], h, y, actn, dgo0, dgo1, dgi0, dgi1)


def _up_step(p0, p1, si, act, bvec, skip, so_next, w_next, fi, fpad):
    def fn(a0, a1, sv, av, hsv, sn, bv, wn):
        agg = (a0 + a1)[:, :fi] * sv
        h = (agg + bv) * av
        u = (h + hsv) * sn
        t = _mm(u, wn)
        if fpad > t.shape[1]:
            t = jnp.concatenate(
                [t, jnp.zeros((t.shape[0], fpad - t.shape[1]), jnp.float32)],
                axis=1)
        return t

    return _rowcall(fn, NP, 6, [fpad], p0, p1, si, act, skip, so_next,
                    bvec, w_next)


def _prep_edges(edge_index, e, n, np_rows, ep):
    pad = ep - e
    pidx = (n + (jnp.arange(pad, dtype=jnp.int32) % (np_rows - n))).astype(jnp.int32)
    s2 = jnp.concatenate([edge_index[0], pidx])
    d2 = jnp.concatenate([edge_index[1], pidx])
    return s2, d2


def _to2d(col):
    return col.reshape(col.shape[0] // 128, 128)


def _tocol(arr2d):
    return arr2d.reshape(arr2d.shape[0] * 128, 1)


def kernel(x, params, edge_index, broadcast, edge_index_hr):
    del broadcast
    p = params
    src2, dst2 = _prep_edges(edge_index, E, N, NP, EP)
    hsrc2, hdst2 = _prep_edges(edge_index_hr, EHR, NHR, NHRP, EPH)

    act0 = (jnp.arange(NP) < N).astype(jnp.float32).reshape(NP, 1)
    acthr = (jnp.arange(NHRP) < NHR).astype(jnp.float32).reshape(NHRP, 1)
    xp = jnp.pad(x, ((0, NP - N), (0, 0)))

    def pnorm(mat):
        return mat * lax.rsqrt(jnp.sum(mat * mat))

    ks = [75000, 56250, 42187, 31640]

    dgo0, dgo1, dgi0, dgi1 = _deg(src2, dst2, act0, NP, 50)

    def t1fn(xv, o0, o1, i0, i1):
        so = lax.rsqrt(jnp.maximum(o0 + o1, 1.0))
        si = lax.rsqrt(jnp.maximum(i0 + i1, 1.0))
        return xv * so, so, si

    table1, so0, si0 = _rowcall(t1fn, NP, 5, [1, 1, 1],
                                xp, dgo0, dgo1, dgi0, dgi1)
    a1 = _agg(src2, dst2, table1, NP, 1, 50)
    h0, y1 = _conv_finish(a1[0], a1[1], si0, act0, p["W_c1"],
                          p["b_c1"].reshape(1, -1), pnorm(p["p1"]), 1)

    hs, ys, acts, sos, sis = [h0], [y1], [act0], [so0], [si0]
    convs = [("c2", 1, 10, 16), ("c3", 10, 14, 16), ("c4", 14, 18, 16),
             ("bn", 18, 18, 18)]
    pools = ["p2", "p3", "p4", None]
    for lvl in range(4):
        actn2d = _topk_mask(_to2d(ys[-1][:, 0:1].reshape(NP)),
                            _to2d(acts[-1].reshape(NP)), ks[lvl])
        actn = _tocol(actn2d)
        no0, no1, ni0, ni1 = _deg(src2, dst2, actn, NP, 50)
        fo_prev = hs[-1].shape[1]
        fpad = 1 if fo_prev == 1 else (16 if fo_prev <= 16 else 18)
        table, so, si = _gate_table(hs[-1], ys[-1], actn,
                                    no0, no1, ni0, ni1, fpad)
        ag = _agg(src2, dst2, table, NP, fpad, 50)
        name, fi, fo, _ = convs[lvl]
        pool = pools[lvl]
        pmat = pnorm(p[pool]) if pool else jnp.eye(fo, dtype=jnp.float32)
        h, y = _conv_finish(ag[0], ag[1], si, actn, p["W_" + name],
                            p["b_" + name].reshape(1, -1), pmat, fi)
        acts.append(actn)
        sos.append(so)
        sis.append(si)
        hs.append(h)
        ys.append(y)

    def mkfn(fo_t, fpad_t):
        def fn(hbnv, hsv, snv, wnv):
            u = (hbnv + hsv) * snv
            t = _mm(u, wnv)
            if fpad_t > t.shape[1]:
                t = jnp.concatenate(
                    [t, jnp.zeros((t.shape[0], fpad_t - t.shape[1]),
                                  jnp.float32)], axis=1)
            return t
        return fn

    tab = _rowcall(mkfn(14, 16), NP, 3, [16], hs[4], hs[3], sos[3], p["W_u1"])
    ag = _agg(src2, dst2, tab, NP, 16, 50)
    tab = _up_step(ag[0], ag[1], sis[3], acts[3], p["b_u1"].reshape(1, -1),
                   hs[2], sos[2], p["W_u2"], 14, 16)
    ag = _agg(src2, dst2, tab, NP, 16, 50)
    tab = _up_step(ag[0], ag[1], sis[2], acts[2], p["b_u2"].reshape(1, -1),
                   hs[1], sos[1], p["W_u3"], 10, 1)
    ag = _agg(src2, dst2, tab, NP, 1, 50)
    tab = _up_step(ag[0], ag[1], sis[1], acts[1], p["b_u3"].reshape(1, -1),
                   hs[0], sos[0], p["W_u4"], 1, 1)
    ag = _agg(src2, dst2, tab, NP, 1, 50)

    def u4fn(a0, a1, sv, av, bv):
        return (a0 + a1) * sv * av + bv * av

    u4 = _rowcall(u4fn, NP, 4, [1], ag[0], ag[1], si0, act0,
                  p["b_u4"].reshape(1, -1))

    neu = jnp.concatenate([u4[:N], u4[:N]], axis=1).reshape(2 * N, 1)
    neu = jnp.pad(neu, ((0, NHRP - NHR), (0, 0)))

    ho0, ho1, hi0, hi1 = _deg(hsrc2, hdst2, acthr, NHRP, 98)

    def thrfn(nv, o0, o1, i0, i1):
        so = lax.rsqrt(jnp.maximum(o0 + o1, 1.0))
        si = lax.rsqrt(jnp.maximum(i0 + i1, 1.0))
        return nv * so, si

    tabbc, sihr = _rowcall(thrfn, NHRP, 5, [1, 1],
                           neu, ho0, ho1, hi0, hi1)
    abc = _agg(hsrc2, hdst2, tabbc, NHRP, 1, 98)

    projs = [(p["projW%d" % j], p["projb%d" % j].reshape(1, -1))
             for j in range(6)]

    def headfn(a0, a1, sv, av, wbc, bbc, *wb):
        h = (a0 + a1) * sv * wbc + bbc
        h = h * av
        for j in range(6):
            h = _mm(h, wb[2 * j]) + wb[2 * j + 1]
            if j < 5:
                h = jnp.where(h >= 0, h, -0.8 * h)
                h = jnp.tanh(h)
        return h

    consts = [p["W_bc"], p["b_bc"].reshape(1, -1)]
    for w, b in projs:
        consts.extend([w, b])
    out = _rowcall(headfn, NHRP, 4, [3], abc[0], abc[1], sihr, acthr, *consts)
    return out[:NHR]
